# Initial kernel scaffold; baseline (speedup 1.0000x reference)
import jax
import jax.numpy as jnp
from jax.experimental import pallas as pl


def kernel(x, edge_index, W_proj, b_proj, W_conv, b_conv):
    raise NotImplementedError("write your pallas kernel here")


# speedup vs baseline: 5.5754x; 5.5754x over previous
import functools

import jax
import jax.numpy as jnp
from jax import lax
from jax.experimental import pallas as pl
from jax.experimental.pallas import tpu as pltpu
from jax.experimental.pallas import tpu_sc as plsc

N = 10000
E = 320000
D = 128

NC = 2
NS = 16
NW = NC * NS
CH = 128
NCH = -(-E // (NW * CH))
EPW = NCH * CH
EPAD = NW * EPW
NPAD = 10240
ZR = 64
RPT = NPAD // NS

_LANES = 16


def _sc_body(x_hbm, src_hbm, dst_hbm, acc_out,
             acc_sh, src_v, dst_v, rows_v, zbuf, gsem, ssem):
    cid = lax.axis_index("c")
    sid = lax.axis_index("s")
    wid = cid * NS + sid

    zero16 = jnp.zeros((_LANES,), jnp.float32)

    def _zrow(r, c):
        for k in range(D // _LANES):
            zbuf[r, pl.ds(k * _LANES, _LANES)] = zero16
        return c
    lax.fori_loop(0, ZR, _zrow, 0)

    def _zacc(i, c):
        pltpu.sync_copy(zbuf, acc_sh.at[pl.ds(sid * RPT + i * ZR, ZR)])
        return c
    lax.fori_loop(0, RPT // ZR, _zacc, 0)

    pltpu.sync_copy(src_hbm.at[wid], src_v)
    pltpu.sync_copy(dst_hbm.at[wid], dst_v)

    plsc.subcore_barrier()

    def _chunk(j, c):
        pltpu.async_copy(x_hbm.at[src_v.at[j]], rows_v, gsem).wait()
        pltpu.async_copy(rows_v, acc_sh.at[dst_v.at[j]], ssem, add=True).wait()
        return c
    lax.fori_loop(0, NCH, _chunk, 0)

    plsc.subcore_barrier()

    pltpu.sync_copy(acc_sh.at[pl.ds(sid * RPT, RPT)],
                    acc_out.at[cid, pl.ds(sid * RPT, RPT)])


_sc_scatter = functools.partial(
    pl.kernel,
    out_type=jax.ShapeDtypeStruct((NC, NPAD, D), jnp.float32),
    mesh=plsc.VectorSubcoreMesh(core_axis_name="c", subcore_axis_name="s"),
    scratch_types=[
        pltpu.VMEM_SHARED((NPAD, D), jnp.float32),
        pltpu.VMEM((NCH, CH), jnp.int32),
        pltpu.VMEM((NCH, CH), jnp.int32),
        pltpu.VMEM((CH, D), jnp.float32),
        pltpu.VMEM((ZR, D), jnp.float32),
        pltpu.SemaphoreType.DMA,
        pltpu.SemaphoreType.DMA,
    ],
)(_sc_body)


DB_R = 8
DB_C = 1280
EPAD2 = 256 * DB_C
DROWS = NPAD // 128


def _deg_body(dst_ref, deg_ref):
    i = pl.program_id(0)

    @pl.when(i == 0)
    def _init():
        deg_ref[...] = jnp.zeros_like(deg_ref)

    d = dst_ref[...]
    acc = deg_ref[...]
    dn = (((1,), (0,)), ((), ()))
    for r in range(DB_R):
        dr = d[r]
        rowid = lax.shift_right_logical(dr, 7)
        colid = lax.bitwise_and(dr, 127)
        oh_r = (lax.broadcasted_iota(jnp.int32, (DROWS, DB_C), 0)
                == rowid[None, :]).astype(jnp.float32)
        oh_c = (colid[:, None]
                == lax.broadcasted_iota(jnp.int32, (DB_C, 128), 1)
                ).astype(jnp.float32)
        acc = acc + lax.dot_general(oh_r, oh_c, dn)
    deg_ref[...] = acc


def _deg_hist(dst2):
    return pl.pallas_call(
        _deg_body,
        grid=(EPAD2 // (DB_R * DB_C),),
        in_specs=[pl.BlockSpec((DB_R, DB_C), lambda i: (i, 0))],
        out_specs=pl.BlockSpec((DROWS, 128), lambda i: (0, 0)),
        out_shape=jax.ShapeDtypeStruct((DROWS, 128), jnp.float32),
    )(dst2)


BN = 1280


def _tc_body(acc_ref, deg_ref, wp_ref, wc_ref, bp_ref, bc_ref, out_ref):
    hi = lax.Precision.HIGHEST
    dn = (((1,), (0,)), ((), ()))
    W = lax.dot_general(wp_ref[...], wc_ref[...], dn, precision=hi)
    bb = lax.dot_general(bp_ref[...], wc_ref[...], dn, precision=hi)
    s = acc_ref[0] + acc_ref[1]
    deg = deg_ref[...]
    mm = lax.dot_general(s, W, dn, precision=hi)
    inv = 1.0 / jnp.maximum(deg, 1.0)
    has = jnp.where(deg > 0.0, 1.0, 0.0)
    out_ref[...] = jnp.maximum(mm * inv + has * bb + bc_ref[...], 0.0)


def _tc_finish(acc, deg_col, W_proj, W_conv, b_proj2, b_conv2):
    return pl.pallas_call(
        _tc_body,
        grid=(NPAD // BN,),
        in_specs=[
            pl.BlockSpec((NC, BN, D), lambda i: (0, i, 0)),
            pl.BlockSpec((BN, 1), lambda i: (i, 0)),
            pl.BlockSpec((D, D), lambda i: (0, 0)),
            pl.BlockSpec((D, D), lambda i: (0, 0)),
            pl.BlockSpec((1, D), lambda i: (0, 0)),
            pl.BlockSpec((1, D), lambda i: (0, 0)),
        ],
        out_specs=pl.BlockSpec((BN, D), lambda i: (i, 0)),
        out_shape=jax.ShapeDtypeStruct((NPAD, D), jnp.float32),
    )(acc, deg_col, W_proj, W_conv, b_proj2, b_conv2)


def kernel(x, edge_index, W_proj, b_proj, W_conv, b_conv):
    src = edge_index[0]
    dst = edge_index[1]
    src_p = jnp.concatenate(
        [src, jnp.zeros((EPAD - E,), jnp.int32)]).reshape(NW, NCH, CH)
    dst_p = jnp.concatenate(
        [dst, jnp.full((EPAD - E,), N, jnp.int32)]).reshape(NW, NCH, CH)
    dst2 = jnp.concatenate(
        [dst, jnp.full((EPAD2 - E,), N, jnp.int32)]).reshape(-1, DB_C)
    acc = _sc_scatter(x, src_p, dst_p)
    deg_col = _deg_hist(dst2).reshape(NPAD, 1)
    out = _tc_finish(acc, deg_col, W_proj, W_conv,
                     b_proj.reshape(1, D), b_conv.reshape(1, D))
    return out[:N]
